# Initial kernel scaffold; baseline (speedup 1.0000x reference)
import jax
import jax.numpy as jnp
from jax.experimental import pallas as pl


def kernel(xmat, mask, Wq, Wk, Wv, Wo, Wg, W1, W2, W3, w1, w2):
    raise NotImplementedError("write your pallas kernel here")


# speedup vs baseline: 1.4262x; 1.4262x over previous
import functools

import jax
import jax.numpy as jnp
from jax.experimental import pallas as pl
from jax.experimental.pallas import tpu as pltpu

DIMS = 1024
EXPNS = 4096
NEXP = 4
BATCH = 128
TJ = 512
NJ = EXPNS // TJ


def _rmsnorm(x, w, eps=1e-6):
    return w * x * jax.lax.rsqrt(jnp.mean(x * x, axis=-1, keepdims=True) + eps)


def _mm_bf16(a, b):
    return jax.lax.dot_general(
        a.astype(jnp.bfloat16), b.astype(jnp.bfloat16),
        (((1,), (1,)), ((), ())), preferred_element_type=jnp.float32)


def _prologue_kernel(x_ref, w1_ref, w2_ref, wv_ref, wo_ref, wg_ref,
                     h_ref, hn_ref, wmat_ref):
    x = x_ref[...]
    xn = _rmsnorm(x, w1_ref[...])
    v = _mm_bf16(xn, wv_ref[...])
    attn = _mm_bf16(v, wo_ref[...])
    h = x + attn
    h_ref[...] = h
    hn = _rmsnorm(h, w2_ref[...])
    hn_ref[...] = hn.astype(jnp.bfloat16)

    g = jax.lax.dot_general(hn, wg_ref[...], (((1,), (1,)), ((), ())),
                            preferred_element_type=jnp.float32,
                            precision=jax.lax.Precision.HIGHEST)
    iota = jax.lax.broadcasted_iota(jnp.int32, (BATCH, NEXP), 1)
    neg = jnp.float32(-1e30)
    m1 = jnp.max(g, axis=1, keepdims=True)
    i1 = jnp.min(jnp.where(g == m1, iota, NEXP), axis=1, keepdims=True)
    g2 = jnp.where(iota == i1, neg, g)
    m2 = jnp.max(g2, axis=1, keepdims=True)
    i2 = jnp.min(jnp.where(g2 == m2, iota, NEXP), axis=1, keepdims=True)
    e2 = jnp.exp(m2 - m1)
    p1 = 1.0 / (1.0 + e2)
    p2 = e2 / (1.0 + e2)
    wmat_ref[...] = (p1 * (iota == i1) + p2 * (iota == i2)).astype(jnp.float32)


def _moe_kernel(hn_ref, wmat_ref, h_ref, w1_ref, w2_ref, w3_ref, out_ref):
    e = pl.program_id(0)
    j = pl.program_id(1)

    @pl.when(jnp.logical_and(e == 0, j == 0))
    def _init():
        out_ref[...] = h_ref[...]

    hn = hn_ref[...]
    a = jax.lax.dot_general(hn, w1_ref[0].astype(jnp.bfloat16),
                            (((1,), (1,)), ((), ())),
                            preferred_element_type=jnp.float32)
    b = jax.lax.dot_general(hn, w2_ref[0].astype(jnp.bfloat16),
                            (((1,), (1,)), ((), ())),
                            preferred_element_type=jnp.float32)
    u = (a * jax.lax.logistic(a)) * b
    iota = jax.lax.broadcasted_iota(jnp.int32, (BATCH, NEXP), 1)
    wcol = jnp.sum(wmat_ref[...] * (iota == e), axis=1, keepdims=True)
    ub = (u * wcol).astype(jnp.bfloat16)
    part = jax.lax.dot_general(ub, w3_ref[0].astype(jnp.bfloat16),
                               (((1,), (1,)), ((), ())),
                               preferred_element_type=jnp.float32)
    out_ref[...] += part


@functools.partial(jax.jit, static_argnames=())
def kernel(xmat, mask, Wq, Wk, Wv, Wo, Wg, W1, W2, W3, w1, w2):
    x = xmat[:, 0, :]

    h, hn_bf, wmat = pl.pallas_call(
        _prologue_kernel,
        out_shape=(
            jax.ShapeDtypeStruct((BATCH, DIMS), jnp.float32),
            jax.ShapeDtypeStruct((BATCH, DIMS), jnp.bfloat16),
            jax.ShapeDtypeStruct((BATCH, NEXP), jnp.float32),
        ),
    )(x, w1.reshape(1, DIMS), w2.reshape(1, DIMS), Wv, Wo, Wg)

    out = pl.pallas_call(
        _moe_kernel,
        grid=(NEXP, NJ),
        in_specs=[
            pl.BlockSpec((BATCH, DIMS), lambda e, j: (0, 0)),
            pl.BlockSpec((BATCH, NEXP), lambda e, j: (0, 0)),
            pl.BlockSpec((BATCH, DIMS), lambda e, j: (0, 0)),
            pl.BlockSpec((1, TJ, DIMS), lambda e, j: (e, j, 0)),
            pl.BlockSpec((1, TJ, DIMS), lambda e, j: (e, j, 0)),
            pl.BlockSpec((1, DIMS, TJ), lambda e, j: (e, 0, j)),
        ],
        out_specs=pl.BlockSpec((BATCH, DIMS), lambda e, j: (0, 0)),
        out_shape=jax.ShapeDtypeStruct((BATCH, DIMS), jnp.float32),
        compiler_params=pltpu.CompilerParams(
            dimension_semantics=("arbitrary", "arbitrary")),
    )(hn_bf, wmat, h, W1, W2, W3)

    return out[:, None, :]


# speedup vs baseline: 1.5702x; 1.1010x over previous
import functools

import jax
import jax.numpy as jnp
from jax.experimental import pallas as pl
from jax.experimental.pallas import tpu as pltpu

DIMS = 1024
EXPNS = 4096
NEXP = 4
BATCH = 128
TJ = 1024
NJ = EXPNS // TJ


def _rmsnorm(x, w, eps=1e-6):
    return w * x * jax.lax.rsqrt(jnp.mean(x * x, axis=-1, keepdims=True) + eps)


def _mm_bf16(a, b):
    return jax.lax.dot_general(
        a.astype(jnp.bfloat16), b.astype(jnp.bfloat16),
        (((1,), (1,)), ((), ())), preferred_element_type=jnp.float32)


def _prologue_kernel(x_ref, w1_ref, w2_ref, wv_ref, wo_ref, wg_ref,
                     h_ref, hn_ref, wmat_ref):
    x = x_ref[...]
    xn = _rmsnorm(x, w1_ref[...])
    v = _mm_bf16(xn, wv_ref[...])
    attn = _mm_bf16(v, wo_ref[...])
    h = x + attn
    h_ref[...] = h
    hn = _rmsnorm(h, w2_ref[...])
    hn_ref[...] = hn.astype(jnp.bfloat16)

    g = jax.lax.dot_general(hn, wg_ref[...], (((1,), (1,)), ((), ())),
                            preferred_element_type=jnp.float32,
                            precision=jax.lax.Precision.HIGHEST)
    iota = jax.lax.broadcasted_iota(jnp.int32, (BATCH, NEXP), 1)
    neg = jnp.float32(-1e30)
    m1 = jnp.max(g, axis=1, keepdims=True)
    i1 = jnp.min(jnp.where(g == m1, iota, NEXP), axis=1, keepdims=True)
    g2 = jnp.where(iota == i1, neg, g)
    m2 = jnp.max(g2, axis=1, keepdims=True)
    i2 = jnp.min(jnp.where(g2 == m2, iota, NEXP), axis=1, keepdims=True)
    e2 = jnp.exp(m2 - m1)
    p1 = 1.0 / (1.0 + e2)
    p2 = e2 / (1.0 + e2)
    wmat_ref[...] = (p1 * (iota == i1) + p2 * (iota == i2)).astype(jnp.float32)


def _moe_kernel(hn_ref, wmat_ref, h_ref, w1_ref, w2_ref, w3_ref, out_ref):
    e = pl.program_id(0)
    j = pl.program_id(1)

    @pl.when(jnp.logical_and(e == 0, j == 0))
    def _init():
        out_ref[...] = h_ref[...]

    hn = hn_ref[...]
    a = jax.lax.dot_general(hn, w1_ref[0].astype(jnp.bfloat16),
                            (((1,), (1,)), ((), ())),
                            preferred_element_type=jnp.float32)
    b = jax.lax.dot_general(hn, w2_ref[0].astype(jnp.bfloat16),
                            (((1,), (1,)), ((), ())),
                            preferred_element_type=jnp.float32)
    u = (a * jax.lax.logistic(a)) * b
    iota = jax.lax.broadcasted_iota(jnp.int32, (BATCH, NEXP), 1)
    wcol = jnp.sum(wmat_ref[...] * (iota == e), axis=1, keepdims=True)
    ub = (u * wcol).astype(jnp.bfloat16)
    part = jax.lax.dot_general(ub, w3_ref[0].astype(jnp.bfloat16),
                               (((1,), (1,)), ((), ())),
                               preferred_element_type=jnp.float32)
    out_ref[...] += part


@functools.partial(jax.jit, static_argnames=())
def kernel(xmat, mask, Wq, Wk, Wv, Wo, Wg, W1, W2, W3, w1, w2):
    x = xmat[:, 0, :]

    h, hn_bf, wmat = pl.pallas_call(
        _prologue_kernel,
        out_shape=(
            jax.ShapeDtypeStruct((BATCH, DIMS), jnp.float32),
            jax.ShapeDtypeStruct((BATCH, DIMS), jnp.bfloat16),
            jax.ShapeDtypeStruct((BATCH, NEXP), jnp.float32),
        ),
    )(x, w1.reshape(1, DIMS), w2.reshape(1, DIMS), Wv, Wo, Wg)

    out = pl.pallas_call(
        _moe_kernel,
        grid=(NEXP, NJ),
        in_specs=[
            pl.BlockSpec((BATCH, DIMS), lambda e, j: (0, 0)),
            pl.BlockSpec((BATCH, NEXP), lambda e, j: (0, 0)),
            pl.BlockSpec((BATCH, DIMS), lambda e, j: (0, 0)),
            pl.BlockSpec((1, TJ, DIMS), lambda e, j: (e, j, 0)),
            pl.BlockSpec((1, TJ, DIMS), lambda e, j: (e, j, 0)),
            pl.BlockSpec((1, DIMS, TJ), lambda e, j: (e, 0, j)),
        ],
        out_specs=pl.BlockSpec((BATCH, DIMS), lambda e, j: (0, 0)),
        out_shape=jax.ShapeDtypeStruct((BATCH, DIMS), jnp.float32),
        compiler_params=pltpu.CompilerParams(
            dimension_semantics=("arbitrary", "arbitrary")),
    )(hn_bf, wmat, h, W1, W2, W3)

    return out[:, None, :]


# speedup vs baseline: 1.6172x; 1.0299x over previous
import functools

import jax
import jax.numpy as jnp
from jax.experimental import pallas as pl
from jax.experimental.pallas import tpu as pltpu

DIMS = 1024
EXPNS = 4096
NEXP = 4
BATCH = 128
TJ = 1024
NJ = EXPNS // TJ
GRID = 1 + NEXP * NJ


def _rmsnorm(x, w, eps=1e-6):
    return w * x * jax.lax.rsqrt(jnp.mean(x * x, axis=-1, keepdims=True) + eps)


def _mm_bf16(a, b):
    return jax.lax.dot_general(
        a.astype(jnp.bfloat16), b.astype(jnp.bfloat16),
        (((1,), (1,)), ((), ())), preferred_element_type=jnp.float32)


def _mm_fast(a, b):
    return jax.lax.dot_general(a, b, (((1,), (1,)), ((), ())),
                               preferred_element_type=jnp.float32,
                               precision=jax.lax.Precision.DEFAULT)


def _block_kernel(x_ref, w1_ref, w2_ref, wv_ref, wo_ref, wg_ref,
                  w1e_ref, w2e_ref, w3e_ref, out_ref,
                  h_s, hn_s, wmat_s):
    i = pl.program_id(0)

    @pl.when(i == 0)
    def _prologue():
        x = x_ref[...]
        xn = _rmsnorm(x, w1_ref[...])
        v = _mm_bf16(xn, wv_ref[...])
        attn = _mm_bf16(v, wo_ref[...])
        h = x + attn
        h_s[...] = h
        out_ref[...] = h
        hn = _rmsnorm(h, w2_ref[...])
        hn_s[...] = hn

        g = jax.lax.dot_general(hn, wg_ref[...], (((1,), (1,)), ((), ())),
                                preferred_element_type=jnp.float32,
                                precision=jax.lax.Precision.HIGHEST)
        iota = jax.lax.broadcasted_iota(jnp.int32, (BATCH, NEXP), 1)
        neg = jnp.float32(-1e30)
        m1 = jnp.max(g, axis=1, keepdims=True)
        i1 = jnp.min(jnp.where(g == m1, iota, NEXP), axis=1, keepdims=True)
        g2 = jnp.where(iota == i1, neg, g)
        m2 = jnp.max(g2, axis=1, keepdims=True)
        i2 = jnp.min(jnp.where(g2 == m2, iota, NEXP), axis=1, keepdims=True)
        e2 = jnp.exp(m2 - m1)
        p1 = 1.0 / (1.0 + e2)
        p2 = e2 / (1.0 + e2)
        wmat_s[...] = (p1 * (iota == i1) + p2 * (iota == i2)).astype(
            jnp.float32)

    @pl.when(i > 0)
    def _moe_tile():
        e = (i - 1) // NJ
        hn = hn_s[...]
        a = _mm_fast(hn, w1e_ref[0])
        b = _mm_fast(hn, w2e_ref[0])
        u = (a * jax.lax.logistic(a)) * b
        iota = jax.lax.broadcasted_iota(jnp.int32, (BATCH, NEXP), 1)
        wcol = jnp.sum(wmat_s[...] * (iota == e), axis=1, keepdims=True)
        out_ref[...] += _mm_fast(u * wcol, w3e_ref[0])


def _widx(i):
    k = jnp.maximum(i - 1, 0)
    return k // NJ, k % NJ


@functools.partial(jax.jit, static_argnames=())
def kernel(xmat, mask, Wq, Wk, Wv, Wo, Wg, W1, W2, W3, w1, w2):
    x = xmat[:, 0, :]

    out = pl.pallas_call(
        _block_kernel,
        grid=(GRID,),
        in_specs=[
            pl.BlockSpec((BATCH, DIMS), lambda i: (0, 0)),
            pl.BlockSpec((1, DIMS), lambda i: (0, 0)),
            pl.BlockSpec((1, DIMS), lambda i: (0, 0)),
            pl.BlockSpec((DIMS, DIMS), lambda i: (0, 0)),
            pl.BlockSpec((DIMS, DIMS), lambda i: (0, 0)),
            pl.BlockSpec((NEXP, DIMS), lambda i: (0, 0)),
            pl.BlockSpec((1, TJ, DIMS), lambda i: (*_widx(i), 0)),
            pl.BlockSpec((1, TJ, DIMS), lambda i: (*_widx(i), 0)),
            pl.BlockSpec((1, DIMS, TJ), lambda i: (_widx(i)[0], 0, _widx(i)[1])),
        ],
        out_specs=pl.BlockSpec((BATCH, DIMS), lambda i: (0, 0)),
        out_shape=jax.ShapeDtypeStruct((BATCH, DIMS), jnp.float32),
        scratch_shapes=[
            pltpu.VMEM((BATCH, DIMS), jnp.float32),
            pltpu.VMEM((BATCH, DIMS), jnp.float32),
            pltpu.VMEM((BATCH, NEXP), jnp.float32),
        ],
        compiler_params=pltpu.CompilerParams(
            dimension_semantics=("arbitrary",)),
    )(x, w1.reshape(1, DIMS), w2.reshape(1, DIMS), Wv, Wo, Wg, W1, W2, W3)

    return out[:, None, :]
